# baseline (speedup 1.0000x reference)
import functools

import jax
import jax.numpy as jnp
from jax import lax
from jax.experimental import pallas as pl
from jax.experimental.pallas import tpu as pltpu
from jax.experimental.pallas import tpu_sc as plsc

_B, _H, _W = 4, 256, 512
_D = 192
_PH, _PW = 64, 128
_CHUNK = 16
_NACC = 128
_NSUB = 32

_TC_NB = 0
_SC_ROW0 = _TC_NB * _PH
_SC_ROWS = (_B - _TC_NB) * _PH
_ROWS_PER_SUB = _SC_ROWS // _NSUB

_MAX_DISP = 192.0
_EPS = 1e-6


def _robust(diff):
    x = diff * 2.0
    x = x * x * (1.0 / 1.2) + 1.0
    x = jnp.exp(0.4 * jnp.log(x))
    return (x - 1.0) * 1.5


def _pool_cols(tr):
    wi = jax.lax.broadcasted_iota(jnp.int32, (_W, _PW), 0)
    ci = jax.lax.broadcasted_iota(jnp.int32, (_W, _PW), 1)
    pooled = None
    for k in range(4):
        sk = (wi == 4 * ci + k).astype(jnp.float32)
        pk = jax.lax.dot(tr, sk, preferred_element_type=jnp.float32)
        pooled = pk if pooled is None else jnp.maximum(pooled, pk)
    return pooled


def _volume_terms(pooled, vol, ph):
    mask = (pooled < _MAX_DISP) & (pooled > 0.001)
    mf = mask.astype(jnp.float32)
    e = 1.0 + jnp.maximum(0.0, pooled - (_MAX_DISP - 1.0))
    diota = jax.lax.broadcasted_iota(jnp.int32, (_D, ph, _PW), 0)
    absd = jnp.abs(diota.astype(jnp.float32) - pooled[None])
    phi = jnp.sum(vol * jnp.maximum(0.0, e[None] - absd), axis=0)
    minv = jnp.min(jnp.where(absd > 1.5, vol, jnp.inf), axis=0)
    sm = jnp.sum(mf)
    gt = jnp.sum(phi * mf)
    nm = jnp.sum(jnp.maximum(1.0 - minv, 0.0) * mf)
    return sm, gt, nm


def _pixel_kernel(t_ref,
                  p0, p1, p2, p3, p4, p5,
                  c0, c1, c2, c3,
                  dxy_ref,
                  s0, s1, s2, s3, s4, s5,
                  *refs):
    vol_ref = refs[0] if _TC_NB else None
    out_ref = refs[-1]
    step = pl.program_id(0)
    t = t_ref[...]
    mask = (t < _MAX_DISP) & (t > 0.001)
    mf = mask.astype(jnp.float32)

    accs = []
    accs.append(jnp.sum(mf))

    preds = (p0, p1, p2, p3, p4, p5)
    diffs = [jnp.abs(p[...] - t) for p in preds]
    rl = 0.0
    for d in diffs:
        rl = rl + jnp.sum(_robust(d) * mf)
    accs.append(rl)

    s_num, s_den = [], []
    for i, s in enumerate((s0, s1, s2, s3, s4, s5)):
        m = mf * (diffs[i] < 1.0).astype(jnp.float32)
        tot = 0.0
        for b in range(_B):
            cross = 0.0
            for b2 in range(_B):
                cross = cross + (jnp.abs(dxy_ref[b, 0] - s[b2, 0])
                                 + jnp.abs(dxy_ref[b, 1] - s[b2, 1]))
            tot = tot + jnp.sum(m[b] * cross)
        s_num.append(tot)
        s_den.append(jnp.sum(m))
    accs.extend(s_num)
    accs.extend(s_den)

    confs = (c0, c1, c2, c3)
    conf_diff_idx = (1, 2, 4, 5)
    c_num, c_den = [], []
    for cr, di in zip(confs, conf_diff_idx):
        d = diffs[di]
        closer = (d < 1.0).astype(jnp.float32)
        further = (d > 1.5).astype(jnp.float32)
        sel = closer + further
        m = mf * sel
        cv = cr[...]
        loss = jnp.maximum(1.0 - cv, 0.0) * closer + jnp.maximum(cv, 0.0) * further
        c_num.append(jnp.sum(loss * m))
        c_den.append(jnp.sum(m))
    accs.extend(c_num)
    accs.extend(c_den)

    ph = _CHUNK // 4
    sm = gt = nm = 0.0
    for tb in range(_TC_NB):
        tr = jnp.max(t[tb].reshape(ph, 4, _W), axis=1)
        pooled = _pool_cols(tr)
        s_, g_, n_ = _volume_terms(pooled, vol_ref[tb], ph)
        sm, gt, nm = sm + s_, gt + g_, nm + n_
    accs.extend([sm, gt, nm])

    lane = jax.lax.broadcasted_iota(jnp.int32, (1, _NACC), 1)
    row = jnp.zeros((1, _NACC), jnp.float32)
    for i, v in enumerate(accs):
        row = jnp.where(lane == i, v, row)

    @pl.when(step == 0)
    def _():
        out_ref[...] = row

    @pl.when(step != 0)
    def _():
        out_ref[...] = out_ref[...] + row


def _vgather(v, idx):
    return lax.gather(
        v, idx[:, None],
        dimension_numbers=lax.GatherDimensionNumbers(
            offset_dims=(), collapsed_slice_dims=(0,), start_index_map=(0,)),
        slice_sizes=(1,),
        mode=lax.GatherScatterMode.PROMISE_IN_BOUNDS)


_sc_mesh = plsc.VectorSubcoreMesh(core_axis_name="c", subcore_axis_name="s")


@functools.partial(
    pl.kernel,
    out_type=jax.ShapeDtypeStruct((3, _NSUB, 16), jnp.float32),
    mesh=_sc_mesh,
    scratch_types=[
        pltpu.VMEM((2, _D, _PW), jnp.float32),
        pltpu.VMEM((2, 4, _W), jnp.float32),
        pltpu.VMEM((3, 16), jnp.float32),
        pltpu.SemaphoreType.DMA,
        pltpu.SemaphoreType.DMA,
        pltpu.SemaphoreType.DMA,
        pltpu.SemaphoreType.DMA,
    ],
)
def _sc_volume(vol_hbm, t_hbm, out_hbm, vol_v, trow_v, acc_v,
               sem0, sem1, tsem0, tsem1):
    wid = lax.axis_index("s") * 2 + lax.axis_index("c")
    inf16 = jnp.full((16,), jnp.inf, jnp.float32)
    zero16 = jnp.zeros((16,), jnp.float32)
    acc_sm = zero16
    acc_gt = zero16
    acc_nm = zero16
    sems = (sem0, sem1)
    tsems = (tsem0, tsem1)
    iota16 = lax.iota(jnp.int32, 16)
    jmod4 = jnp.bitwise_and(iota16, 3)
    jdiv4 = jnp.right_shift(iota16, 2)
    shuffle_idx = [4 * jmod4 + k for k in range(4)]
    chunk_masks = [jdiv4 == m for m in range(4)]

    base = wid * _ROWS_PER_SUB

    def _slab_copy(rr, buf):
        row = _SC_ROW0 + base + rr
        h = pltpu.async_copy(vol_hbm.at[row // _PH, :, row % _PH, :],
                             vol_v.at[buf], sems[buf])
        hs = [pltpu.async_copy(t_hbm.at[row // _PH, 4 * (row % _PH) + j, :],
                               trow_v.at[buf, j], tsems[buf])
              for j in range(4)]
        return [h] + hs

    pending = _slab_copy(0, 0)
    for rr in range(_ROWS_PER_SUB):
        cur = rr % 2
        for h in pending:
            h.wait()
        if rr + 1 < _ROWS_PER_SUB:
            pending = _slab_copy(rr + 1, 1 - cur)

        groups = []
        for g in range(8):
            chunks = []
            for m in range(4):
                c = trow_v[cur, 0, pl.ds(64 * g + 16 * m, 16)]
                for j in range(1, 4):
                    c = jnp.maximum(c, trow_v[cur, j, pl.ds(64 * g + 16 * m, 16)])
                chunks.append(c)
            t16 = None
            for k in range(4):
                tap = None
                for m in range(4):
                    gk = _vgather(chunks[m], shuffle_idx[k])
                    tap = gk if tap is None else jnp.where(chunk_masks[m], gk, tap)
                t16 = tap if t16 is None else jnp.maximum(t16, tap)
            mask = (t16 < _MAX_DISP) & (t16 > 0.001)
            mf = jnp.where(mask, 1.0, 0.0)
            e16 = 1.0 + jnp.maximum(0.0, t16 - (_MAX_DISP - 1.0))
            groups.append((t16, mf, e16))

        def body(d, carry):
            d_f = jnp.full((16,), d, jnp.float32)
            out = []
            for g in range(8):
                t16, _, e16 = groups[g]
                mv, ph = carry[2 * g], carry[2 * g + 1]
                v = vol_v[cur, d, pl.ds(g * 16, 16)]
                absd = jnp.abs(d_f - t16)
                keep = absd > 1.5
                mv = jnp.minimum(mv, jnp.where(keep, v, inf16))
                ph = ph + v * jnp.maximum(0.0, e16 - absd)
                out.extend((mv, ph))
            return tuple(out)

        carry = lax.fori_loop(0, _D, body, (inf16, zero16) * 8)

        for g in range(8):
            t16, mf, _ = groups[g]
            mv, ph = carry[2 * g], carry[2 * g + 1]
            acc_sm = acc_sm + mf
            acc_gt = acc_gt + ph * mf
            acc_nm = acc_nm + jnp.maximum(1.0 - mv, 0.0) * mf

    acc_v[0, :] = acc_sm
    acc_v[1, :] = acc_gt
    acc_v[2, :] = acc_nm
    pltpu.sync_copy(acc_v, out_hbm.at[:, wid, :])


def _combine_kernel(acc_ref, sc_ref, out_ref):
    sm = jnp.sum(sc_ref[0]) + acc_ref[0, 22]
    gt = jnp.sum(sc_ref[1]) + acc_ref[0, 23]
    nm = jnp.sum(sc_ref[2]) + acc_ref[0, 24]
    scale_l = acc_ref[0, 1] / (acc_ref[0, 0] + _EPS)
    slant_l = 0.0
    for i in range(6):
        slant_l = slant_l + acc_ref[0, 2 + i] / (acc_ref[0, 8 + i] + _EPS)
    conf_l = 0.0
    for i in range(4):
        conf_l = conf_l + acc_ref[0, 14 + i] / (acc_ref[0, 18 + i] + _EPS)
    init_l = (gt + nm) / (sm + _EPS)
    out_ref[0, 0] = scale_l + init_l + slant_l + conf_l


def kernel(preds_0, preds_1, preds_2, preds_coarse_0, preds_coarse_1,
           preds_coarse_2, slant_0, slant_1, slant_2, slant_coarse_0,
           slant_coarse_1, slant_coarse_2, conf_0, conf_1, conf_coarse_0,
           conf_coarse_1, volume_0, target, dxygt):
    sc_part = _sc_volume(volume_0, target)

    pix_inputs = [target,
                  preds_0, preds_1, preds_2,
                  preds_coarse_0, preds_coarse_1, preds_coarse_2,
                  conf_0, conf_1, conf_coarse_0, conf_coarse_1,
                  dxygt,
                  slant_0, slant_1, slant_2,
                  slant_coarse_0, slant_coarse_1, slant_coarse_2]

    n_steps = _H // _CHUNK
    in_spec3 = pl.BlockSpec((_B, _CHUNK, _W), lambda i: (0, i, 0))
    in_spec4 = pl.BlockSpec((_B, 2, _CHUNK, _W), lambda i: (0, 0, i, 0))
    specs = [in_spec3] * 11 + [in_spec4] * 7
    if _TC_NB:
        specs.append(pl.BlockSpec((_TC_NB, _D, _CHUNK // 4, _PW),
                                  lambda i: (0, 0, i, 0)))
        pix_inputs.append(volume_0)
    acc = pl.pallas_call(
        _pixel_kernel,
        grid=(n_steps,),
        in_specs=specs,
        out_specs=pl.BlockSpec((1, _NACC), lambda i: (0, 0)),
        out_shape=jax.ShapeDtypeStruct((1, _NACC), jnp.float32),
    )(*pix_inputs)

    out = pl.pallas_call(
        _combine_kernel,
        in_specs=[pl.BlockSpec(memory_space=pltpu.SMEM),
                  pl.BlockSpec(memory_space=pltpu.VMEM)],
        out_specs=pl.BlockSpec(memory_space=pltpu.SMEM),
        out_shape=jax.ShapeDtypeStruct((1, 1), jnp.float32),
    )(acc, sc_part)

    return out[0, 0]


# speedup vs baseline: 1.0034x; 1.0034x over previous
import functools

import jax
import jax.numpy as jnp
from jax import lax
from jax.experimental import pallas as pl
from jax.experimental.pallas import tpu as pltpu
from jax.experimental.pallas import tpu_sc as plsc

_B, _H, _W = 4, 256, 512
_D = 192
_PH, _PW = 64, 128
_CHUNK = 32
_NACC = 128
_NSUB = 32

_TC_NB = 0
_SC_ROW0 = _TC_NB * _PH
_SC_ROWS = (_B - _TC_NB) * _PH
_ROWS_PER_SUB = _SC_ROWS // _NSUB

_MAX_DISP = 192.0
_EPS = 1e-6


def _robust(diff):
    x = diff * 2.0
    x = x * x * (1.0 / 1.2) + 1.0
    x = jnp.exp(0.4 * jnp.log(x))
    return (x - 1.0) * 1.5


def _pool_cols(tr):
    wi = jax.lax.broadcasted_iota(jnp.int32, (_W, _PW), 0)
    ci = jax.lax.broadcasted_iota(jnp.int32, (_W, _PW), 1)
    pooled = None
    for k in range(4):
        sk = (wi == 4 * ci + k).astype(jnp.float32)
        pk = jax.lax.dot(tr, sk, preferred_element_type=jnp.float32)
        pooled = pk if pooled is None else jnp.maximum(pooled, pk)
    return pooled


def _volume_terms(pooled, vol, ph):
    mask = (pooled < _MAX_DISP) & (pooled > 0.001)
    mf = mask.astype(jnp.float32)
    e = 1.0 + jnp.maximum(0.0, pooled - (_MAX_DISP - 1.0))
    diota = jax.lax.broadcasted_iota(jnp.int32, (_D, ph, _PW), 0)
    absd = jnp.abs(diota.astype(jnp.float32) - pooled[None])
    phi = jnp.sum(vol * jnp.maximum(0.0, e[None] - absd), axis=0)
    minv = jnp.min(jnp.where(absd > 1.5, vol, jnp.inf), axis=0)
    sm = jnp.sum(mf)
    gt = jnp.sum(phi * mf)
    nm = jnp.sum(jnp.maximum(1.0 - minv, 0.0) * mf)
    return sm, gt, nm


def _pixel_kernel(t_ref,
                  p0, p1, p2, p3, p4, p5,
                  c0, c1, c2, c3,
                  dxy_ref,
                  s0, s1, s2, s3, s4, s5,
                  *refs):
    vol_ref = refs[0] if _TC_NB else None
    out_ref = refs[-1]
    step = pl.program_id(0)
    t = t_ref[...]
    mask = (t < _MAX_DISP) & (t > 0.001)
    mf = mask.astype(jnp.float32)

    accs = []
    accs.append(jnp.sum(mf))

    preds = (p0, p1, p2, p3, p4, p5)
    diffs = [jnp.abs(p[...] - t) for p in preds]
    rl = 0.0
    for d in diffs:
        rl = rl + jnp.sum(_robust(d) * mf)
    accs.append(rl)

    s_num, s_den = [], []
    for i, s in enumerate((s0, s1, s2, s3, s4, s5)):
        m = mf * (diffs[i] < 1.0).astype(jnp.float32)
        tot = 0.0
        for b in range(_B):
            cross = 0.0
            for b2 in range(_B):
                cross = cross + (jnp.abs(dxy_ref[b, 0] - s[b2, 0])
                                 + jnp.abs(dxy_ref[b, 1] - s[b2, 1]))
            tot = tot + jnp.sum(m[b] * cross)
        s_num.append(tot)
        s_den.append(jnp.sum(m))
    accs.extend(s_num)
    accs.extend(s_den)

    confs = (c0, c1, c2, c3)
    conf_diff_idx = (1, 2, 4, 5)
    c_num, c_den = [], []
    for cr, di in zip(confs, conf_diff_idx):
        d = diffs[di]
        closer = (d < 1.0).astype(jnp.float32)
        further = (d > 1.5).astype(jnp.float32)
        sel = closer + further
        m = mf * sel
        cv = cr[...]
        loss = jnp.maximum(1.0 - cv, 0.0) * closer + jnp.maximum(cv, 0.0) * further
        c_num.append(jnp.sum(loss * m))
        c_den.append(jnp.sum(m))
    accs.extend(c_num)
    accs.extend(c_den)

    ph = _CHUNK // 4
    sm = gt = nm = 0.0
    for tb in range(_TC_NB):
        tr = jnp.max(t[tb].reshape(ph, 4, _W), axis=1)
        pooled = _pool_cols(tr)
        s_, g_, n_ = _volume_terms(pooled, vol_ref[tb], ph)
        sm, gt, nm = sm + s_, gt + g_, nm + n_
    accs.extend([sm, gt, nm])

    lane = jax.lax.broadcasted_iota(jnp.int32, (1, _NACC), 1)
    row = jnp.zeros((1, _NACC), jnp.float32)
    for i, v in enumerate(accs):
        row = jnp.where(lane == i, v, row)

    @pl.when(step == 0)
    def _():
        out_ref[...] = row

    @pl.when(step != 0)
    def _():
        out_ref[...] = out_ref[...] + row


def _vgather(v, idx):
    return lax.gather(
        v, idx[:, None],
        dimension_numbers=lax.GatherDimensionNumbers(
            offset_dims=(), collapsed_slice_dims=(0,), start_index_map=(0,)),
        slice_sizes=(1,),
        mode=lax.GatherScatterMode.PROMISE_IN_BOUNDS)


_sc_mesh = plsc.VectorSubcoreMesh(core_axis_name="c", subcore_axis_name="s")


@functools.partial(
    pl.kernel,
    out_type=jax.ShapeDtypeStruct((3, _NSUB, 16), jnp.float32),
    mesh=_sc_mesh,
    scratch_types=[
        pltpu.VMEM((2, _D, _PW), jnp.float32),
        pltpu.VMEM((2, 4, _W), jnp.float32),
        pltpu.VMEM((3, 16), jnp.float32),
        pltpu.SemaphoreType.DMA,
        pltpu.SemaphoreType.DMA,
        pltpu.SemaphoreType.DMA,
        pltpu.SemaphoreType.DMA,
    ],
)
def _sc_volume(vol_hbm, t_hbm, out_hbm, vol_v, trow_v, acc_v,
               sem0, sem1, tsem0, tsem1):
    wid = lax.axis_index("s") * 2 + lax.axis_index("c")
    inf16 = jnp.full((16,), jnp.inf, jnp.float32)
    zero16 = jnp.zeros((16,), jnp.float32)
    acc_sm = zero16
    acc_gt = zero16
    acc_nm = zero16
    sems = (sem0, sem1)
    tsems = (tsem0, tsem1)
    iota16 = lax.iota(jnp.int32, 16)
    jmod4 = jnp.bitwise_and(iota16, 3)
    jdiv4 = jnp.right_shift(iota16, 2)
    shuffle_idx = [4 * jmod4 + k for k in range(4)]
    chunk_masks = [jdiv4 == m for m in range(4)]

    base = wid * _ROWS_PER_SUB

    def _slab_copy(rr, buf):
        row = _SC_ROW0 + base + rr
        h = pltpu.async_copy(vol_hbm.at[row // _PH, :, row % _PH, :],
                             vol_v.at[buf], sems[buf])
        hs = [pltpu.async_copy(t_hbm.at[row // _PH, 4 * (row % _PH) + j, :],
                               trow_v.at[buf, j], tsems[buf])
              for j in range(4)]
        return [h] + hs

    pending = _slab_copy(0, 0)
    for rr in range(_ROWS_PER_SUB):
        cur = rr % 2
        for h in pending:
            h.wait()
        if rr + 1 < _ROWS_PER_SUB:
            pending = _slab_copy(rr + 1, 1 - cur)

        groups = []
        for g in range(8):
            chunks = []
            for m in range(4):
                c = trow_v[cur, 0, pl.ds(64 * g + 16 * m, 16)]
                for j in range(1, 4):
                    c = jnp.maximum(c, trow_v[cur, j, pl.ds(64 * g + 16 * m, 16)])
                chunks.append(c)
            t16 = None
            for k in range(4):
                tap = None
                for m in range(4):
                    gk = _vgather(chunks[m], shuffle_idx[k])
                    tap = gk if tap is None else jnp.where(chunk_masks[m], gk, tap)
                t16 = tap if t16 is None else jnp.maximum(t16, tap)
            mask = (t16 < _MAX_DISP) & (t16 > 0.001)
            mf = jnp.where(mask, 1.0, 0.0)
            e16 = 1.0 + jnp.maximum(0.0, t16 - (_MAX_DISP - 1.0))
            groups.append((t16, mf, e16))

        def body(d, carry):
            d_f = jnp.full((16,), d, jnp.float32)
            out = []
            for g in range(8):
                t16, _, e16 = groups[g]
                mv, ph = carry[2 * g], carry[2 * g + 1]
                v = vol_v[cur, d, pl.ds(g * 16, 16)]
                absd = jnp.abs(d_f - t16)
                keep = absd > 1.5
                mv = jnp.minimum(mv, jnp.where(keep, v, inf16))
                ph = ph + v * jnp.maximum(0.0, e16 - absd)
                out.extend((mv, ph))
            return tuple(out)

        carry = lax.fori_loop(0, _D, body, (inf16, zero16) * 8)

        for g in range(8):
            t16, mf, _ = groups[g]
            mv, ph = carry[2 * g], carry[2 * g + 1]
            acc_sm = acc_sm + mf
            acc_gt = acc_gt + ph * mf
            acc_nm = acc_nm + jnp.maximum(1.0 - mv, 0.0) * mf

    acc_v[0, :] = acc_sm
    acc_v[1, :] = acc_gt
    acc_v[2, :] = acc_nm
    pltpu.sync_copy(acc_v, out_hbm.at[:, wid, :])


def _combine_kernel(acc_ref, sc_ref, out_ref):
    sm = jnp.sum(sc_ref[0]) + acc_ref[0, 22]
    gt = jnp.sum(sc_ref[1]) + acc_ref[0, 23]
    nm = jnp.sum(sc_ref[2]) + acc_ref[0, 24]
    scale_l = acc_ref[0, 1] / (acc_ref[0, 0] + _EPS)
    slant_l = 0.0
    for i in range(6):
        slant_l = slant_l + acc_ref[0, 2 + i] / (acc_ref[0, 8 + i] + _EPS)
    conf_l = 0.0
    for i in range(4):
        conf_l = conf_l + acc_ref[0, 14 + i] / (acc_ref[0, 18 + i] + _EPS)
    init_l = (gt + nm) / (sm + _EPS)
    out_ref[0, 0] = scale_l + init_l + slant_l + conf_l


def kernel(preds_0, preds_1, preds_2, preds_coarse_0, preds_coarse_1,
           preds_coarse_2, slant_0, slant_1, slant_2, slant_coarse_0,
           slant_coarse_1, slant_coarse_2, conf_0, conf_1, conf_coarse_0,
           conf_coarse_1, volume_0, target, dxygt):
    sc_part = _sc_volume(volume_0, target)

    pix_inputs = [target,
                  preds_0, preds_1, preds_2,
                  preds_coarse_0, preds_coarse_1, preds_coarse_2,
                  conf_0, conf_1, conf_coarse_0, conf_coarse_1,
                  dxygt,
                  slant_0, slant_1, slant_2,
                  slant_coarse_0, slant_coarse_1, slant_coarse_2]

    n_steps = _H // _CHUNK
    in_spec3 = pl.BlockSpec((_B, _CHUNK, _W), lambda i: (0, i, 0))
    in_spec4 = pl.BlockSpec((_B, 2, _CHUNK, _W), lambda i: (0, 0, i, 0))
    specs = [in_spec3] * 11 + [in_spec4] * 7
    if _TC_NB:
        specs.append(pl.BlockSpec((_TC_NB, _D, _CHUNK // 4, _PW),
                                  lambda i: (0, 0, i, 0)))
        pix_inputs.append(volume_0)
    acc = pl.pallas_call(
        _pixel_kernel,
        grid=(n_steps,),
        in_specs=specs,
        out_specs=pl.BlockSpec((1, _NACC), lambda i: (0, 0)),
        out_shape=jax.ShapeDtypeStruct((1, _NACC), jnp.float32),
    )(*pix_inputs)

    out = pl.pallas_call(
        _combine_kernel,
        in_specs=[pl.BlockSpec(memory_space=pltpu.SMEM),
                  pl.BlockSpec(memory_space=pltpu.VMEM)],
        out_specs=pl.BlockSpec(memory_space=pltpu.SMEM),
        out_shape=jax.ShapeDtypeStruct((1, 1), jnp.float32),
    )(acc, sc_part)

    return out[0, 0]


# speedup vs baseline: 1.0037x; 1.0003x over previous
import functools

import jax
import jax.numpy as jnp
from jax import lax
from jax.experimental import pallas as pl
from jax.experimental.pallas import tpu as pltpu
from jax.experimental.pallas import tpu_sc as plsc

_B, _H, _W = 4, 256, 512
_D = 192
_PH, _PW = 64, 128
_CHUNK = 32
_NACC = 128
_NSUB = 32

_TC_NB = 0
_SC_ROW0 = _TC_NB * _PH
_SC_ROWS = (_B - _TC_NB) * _PH
_ROWS_PER_SUB = _SC_ROWS // _NSUB

_MAX_DISP = 192.0
_EPS = 1e-6


def _robust(diff):
    x = diff * 2.0
    x = x * x * (1.0 / 1.2) + 1.0
    x = jnp.exp(0.4 * jnp.log(x))
    return (x - 1.0) * 1.5


def _pool_cols(tr):
    wi = jax.lax.broadcasted_iota(jnp.int32, (_W, _PW), 0)
    ci = jax.lax.broadcasted_iota(jnp.int32, (_W, _PW), 1)
    pooled = None
    for k in range(4):
        sk = (wi == 4 * ci + k).astype(jnp.float32)
        pk = jax.lax.dot(tr, sk, preferred_element_type=jnp.float32)
        pooled = pk if pooled is None else jnp.maximum(pooled, pk)
    return pooled


def _volume_terms(pooled, vol, ph):
    mask = (pooled < _MAX_DISP) & (pooled > 0.001)
    mf = mask.astype(jnp.float32)
    e = 1.0 + jnp.maximum(0.0, pooled - (_MAX_DISP - 1.0))
    diota = jax.lax.broadcasted_iota(jnp.int32, (_D, ph, _PW), 0)
    absd = jnp.abs(diota.astype(jnp.float32) - pooled[None])
    phi = jnp.sum(vol * jnp.maximum(0.0, e[None] - absd), axis=0)
    minv = jnp.min(jnp.where(absd > 1.5, vol, jnp.inf), axis=0)
    sm = jnp.sum(mf)
    gt = jnp.sum(phi * mf)
    nm = jnp.sum(jnp.maximum(1.0 - minv, 0.0) * mf)
    return sm, gt, nm


def _pixel_kernel(t_ref,
                  p0, p1, p2, p3, p4, p5,
                  c0, c1, c2, c3,
                  dxy_ref,
                  s0, s1, s2, s3, s4, s5,
                  *refs):
    vol_ref = refs[0] if _TC_NB else None
    out_ref = refs[-1]
    step = pl.program_id(0)
    t = t_ref[...]
    mask = (t < _MAX_DISP) & (t > 0.001)
    mf = mask.astype(jnp.float32)

    accs = []
    accs.append(jnp.sum(mf))

    preds = (p0, p1, p2, p3, p4, p5)
    diffs = [jnp.abs(p[...] - t) for p in preds]
    rl = 0.0
    for d in diffs:
        rl = rl + jnp.sum(_robust(d) * mf)
    accs.append(rl)

    s_num, s_den = [], []
    for i, s in enumerate((s0, s1, s2, s3, s4, s5)):
        m = mf * (diffs[i] < 1.0).astype(jnp.float32)
        tot = 0.0
        for b in range(_B):
            cross = 0.0
            for b2 in range(_B):
                cross = cross + (jnp.abs(dxy_ref[b, 0] - s[b2, 0])
                                 + jnp.abs(dxy_ref[b, 1] - s[b2, 1]))
            tot = tot + jnp.sum(m[b] * cross)
        s_num.append(tot)
        s_den.append(jnp.sum(m))
    accs.extend(s_num)
    accs.extend(s_den)

    confs = (c0, c1, c2, c3)
    conf_diff_idx = (1, 2, 4, 5)
    c_num, c_den = [], []
    for cr, di in zip(confs, conf_diff_idx):
        d = diffs[di]
        closer = (d < 1.0).astype(jnp.float32)
        further = (d > 1.5).astype(jnp.float32)
        sel = closer + further
        m = mf * sel
        cv = cr[...]
        loss = jnp.maximum(1.0 - cv, 0.0) * closer + jnp.maximum(cv, 0.0) * further
        c_num.append(jnp.sum(loss * m))
        c_den.append(jnp.sum(m))
    accs.extend(c_num)
    accs.extend(c_den)

    ph = _CHUNK // 4
    sm = gt = nm = 0.0
    for tb in range(_TC_NB):
        tr = jnp.max(t[tb].reshape(ph, 4, _W), axis=1)
        pooled = _pool_cols(tr)
        s_, g_, n_ = _volume_terms(pooled, vol_ref[tb], ph)
        sm, gt, nm = sm + s_, gt + g_, nm + n_
    accs.extend([sm, gt, nm])

    lane = jax.lax.broadcasted_iota(jnp.int32, (1, _NACC), 1)
    row = jnp.zeros((1, _NACC), jnp.float32)
    for i, v in enumerate(accs):
        row = jnp.where(lane == i, v, row)

    @pl.when(step == 0)
    def _():
        out_ref[...] = row

    @pl.when(step != 0)
    def _():
        out_ref[...] = out_ref[...] + row


def _vgather(v, idx):
    return lax.gather(
        v, idx[:, None],
        dimension_numbers=lax.GatherDimensionNumbers(
            offset_dims=(), collapsed_slice_dims=(0,), start_index_map=(0,)),
        slice_sizes=(1,),
        mode=lax.GatherScatterMode.PROMISE_IN_BOUNDS)


@functools.lru_cache(maxsize=1)
def _sc_volume_call():
    mesh = plsc.VectorSubcoreMesh(core_axis_name="c", subcore_axis_name="s")
    return functools.partial(
        pl.kernel,
        out_type=jax.ShapeDtypeStruct((3, _NSUB, 16), jnp.float32),
        mesh=mesh,
        scratch_types=[
            pltpu.VMEM((2, _D, _PW), jnp.float32),
            pltpu.VMEM((2, 4, _W), jnp.float32),
            pltpu.VMEM((3, 16), jnp.float32),
            pltpu.SemaphoreType.DMA,
            pltpu.SemaphoreType.DMA,
            pltpu.SemaphoreType.DMA,
            pltpu.SemaphoreType.DMA,
        ],
    )(_sc_volume)


def _sc_volume(vol_hbm, t_hbm, out_hbm, vol_v, trow_v, acc_v,
               sem0, sem1, tsem0, tsem1):
    wid = lax.axis_index("s") * 2 + lax.axis_index("c")
    inf16 = jnp.full((16,), jnp.inf, jnp.float32)
    zero16 = jnp.zeros((16,), jnp.float32)
    acc_sm = zero16
    acc_gt = zero16
    acc_nm = zero16
    sems = (sem0, sem1)
    tsems = (tsem0, tsem1)
    iota16 = lax.iota(jnp.int32, 16)
    jmod4 = jnp.bitwise_and(iota16, 3)
    jdiv4 = jnp.right_shift(iota16, 2)
    shuffle_idx = [4 * jmod4 + k for k in range(4)]
    chunk_masks = [jdiv4 == m for m in range(4)]

    base = wid * _ROWS_PER_SUB

    def _slab_copy(rr, buf):
        row = _SC_ROW0 + base + rr
        h = pltpu.async_copy(vol_hbm.at[row // _PH, :, row % _PH, :],
                             vol_v.at[buf], sems[buf])
        hs = [pltpu.async_copy(t_hbm.at[row // _PH, 4 * (row % _PH) + j, :],
                               trow_v.at[buf, j], tsems[buf])
              for j in range(4)]
        return [h] + hs

    pending = _slab_copy(0, 0)
    for rr in range(_ROWS_PER_SUB):
        cur = rr % 2
        for h in pending:
            h.wait()
        if rr + 1 < _ROWS_PER_SUB:
            pending = _slab_copy(rr + 1, 1 - cur)

        groups = []
        for g in range(8):
            chunks = []
            for m in range(4):
                c = trow_v[cur, 0, pl.ds(64 * g + 16 * m, 16)]
                for j in range(1, 4):
                    c = jnp.maximum(c, trow_v[cur, j, pl.ds(64 * g + 16 * m, 16)])
                chunks.append(c)
            t16 = None
            for k in range(4):
                tap = None
                for m in range(4):
                    gk = _vgather(chunks[m], shuffle_idx[k])
                    tap = gk if tap is None else jnp.where(chunk_masks[m], gk, tap)
                t16 = tap if t16 is None else jnp.maximum(t16, tap)
            mask = (t16 < _MAX_DISP) & (t16 > 0.001)
            mf = jnp.where(mask, 1.0, 0.0)
            e16 = 1.0 + jnp.maximum(0.0, t16 - (_MAX_DISP - 1.0))
            groups.append((t16, mf, e16))

        def body(d, carry):
            d_f = jnp.full((16,), d, jnp.float32)
            out = []
            for g in range(8):
                t16, _, e16 = groups[g]
                mv, ph = carry[2 * g], carry[2 * g + 1]
                v = vol_v[cur, d, pl.ds(g * 16, 16)]
                absd = jnp.abs(d_f - t16)
                keep = absd > 1.5
                mv = jnp.minimum(mv, jnp.where(keep, v, inf16))
                ph = ph + v * jnp.maximum(0.0, e16 - absd)
                out.extend((mv, ph))
            return tuple(out)

        carry = lax.fori_loop(0, _D, body, (inf16, zero16) * 8)

        for g in range(8):
            t16, mf, _ = groups[g]
            mv, ph = carry[2 * g], carry[2 * g + 1]
            acc_sm = acc_sm + mf
            acc_gt = acc_gt + ph * mf
            acc_nm = acc_nm + jnp.maximum(1.0 - mv, 0.0) * mf

    acc_v[0, :] = acc_sm
    acc_v[1, :] = acc_gt
    acc_v[2, :] = acc_nm
    pltpu.sync_copy(acc_v, out_hbm.at[:, wid, :])


def _combine_kernel(acc_ref, sc_ref, out_ref):
    sm = jnp.sum(sc_ref[0]) + acc_ref[0, 22]
    gt = jnp.sum(sc_ref[1]) + acc_ref[0, 23]
    nm = jnp.sum(sc_ref[2]) + acc_ref[0, 24]
    scale_l = acc_ref[0, 1] / (acc_ref[0, 0] + _EPS)
    slant_l = 0.0
    for i in range(6):
        slant_l = slant_l + acc_ref[0, 2 + i] / (acc_ref[0, 8 + i] + _EPS)
    conf_l = 0.0
    for i in range(4):
        conf_l = conf_l + acc_ref[0, 14 + i] / (acc_ref[0, 18 + i] + _EPS)
    init_l = (gt + nm) / (sm + _EPS)
    out_ref[0, 0] = scale_l + init_l + slant_l + conf_l


def kernel(preds_0, preds_1, preds_2, preds_coarse_0, preds_coarse_1,
           preds_coarse_2, slant_0, slant_1, slant_2, slant_coarse_0,
           slant_coarse_1, slant_coarse_2, conf_0, conf_1, conf_coarse_0,
           conf_coarse_1, volume_0, target, dxygt):
    sc_part = _sc_volume_call()(volume_0, target)

    pix_inputs = [target,
                  preds_0, preds_1, preds_2,
                  preds_coarse_0, preds_coarse_1, preds_coarse_2,
                  conf_0, conf_1, conf_coarse_0, conf_coarse_1,
                  dxygt,
                  slant_0, slant_1, slant_2,
                  slant_coarse_0, slant_coarse_1, slant_coarse_2]

    n_steps = _H // _CHUNK
    in_spec3 = pl.BlockSpec((_B, _CHUNK, _W), lambda i: (0, i, 0))
    in_spec4 = pl.BlockSpec((_B, 2, _CHUNK, _W), lambda i: (0, 0, i, 0))
    specs = [in_spec3] * 11 + [in_spec4] * 7
    if _TC_NB:
        specs.append(pl.BlockSpec((_TC_NB, _D, _CHUNK // 4, _PW),
                                  lambda i: (0, 0, i, 0)))
        pix_inputs.append(volume_0)
    acc = pl.pallas_call(
        _pixel_kernel,
        grid=(n_steps,),
        in_specs=specs,
        out_specs=pl.BlockSpec((1, _NACC), lambda i: (0, 0)),
        out_shape=jax.ShapeDtypeStruct((1, _NACC), jnp.float32),
    )(*pix_inputs)

    out = pl.pallas_call(
        _combine_kernel,
        in_specs=[pl.BlockSpec(memory_space=pltpu.SMEM),
                  pl.BlockSpec(memory_space=pltpu.VMEM)],
        out_specs=pl.BlockSpec(memory_space=pltpu.SMEM),
        out_shape=jax.ShapeDtypeStruct((1, 1), jnp.float32),
    )(acc, sc_part)

    return out[0, 0]
